# baseline (speedup 1.0000x reference)
import functools

import jax
import jax.numpy as jnp
from jax import lax
from jax.experimental import pallas as pl
from jax.experimental.pallas import tpu as pltpu
from jax.experimental.pallas import tpu_sc as plsc

NC = 2
NS = 16
LN = 16


def _make_chase(B, S):
    G = B // LN
    mesh = plsc.VectorSubcoreMesh(core_axis_name="c", subcore_axis_name="s")

    @functools.partial(
        pl.kernel,
        mesh=mesh,
        out_type=jax.ShapeDtypeStruct((B, S), jnp.int32),
        scratch_types=[
            pltpu.VMEM((LN * S,), jnp.int32),
            pltpu.VMEM((LN * S,), jnp.int32),
            pltpu.SemaphoreType.DMA,
        ],
        compiler_params=pltpu.CompilerParams(needs_layout_passes=False),
    )
    def chase(sol_hbm, zer_hbm, vis_hbm, sol_v, vis_v, sem):
        wid = lax.axis_index("s") * NC + lax.axis_index("c")

        @pl.when(wid < G)
        def _():
            base = wid * LN
            for r in range(LN):
                pltpu.async_copy(
                    sol_hbm.at[base + r], sol_v.at[pl.ds(r * S, S)], sem
                )
                pltpu.async_copy(
                    zer_hbm.at[base + r], vis_v.at[pl.ds(r * S, S)], sem
                )
            for r in range(LN):
                pltpu.make_async_copy(
                    sol_hbm.at[base + r], sol_v.at[pl.ds(r * S, S)], sem
                ).wait()
                pltpu.make_async_copy(
                    zer_hbm.at[base + r], vis_v.at[pl.ds(r * S, S)], sem
                ).wait()

            rowbase = lax.iota(jnp.int32, LN) * S
            zeros = jnp.zeros((LN,), jnp.int32)
            ones = jnp.full((LN,), 1, jnp.int32)

            def step(i, carry):
                del i
                pre, stepv = carry
                cur = plsc.load_gather(sol_v, [rowbase + pre])
                plsc.store_scatter(vis_v, [rowbase + cur], stepv)
                return (cur, stepv + ones)

            lax.fori_loop(0, S, step, (zeros, ones), unroll=8)

            for r in range(LN):
                pltpu.async_copy(
                    vis_v.at[pl.ds(r * S, S)], vis_hbm.at[base + r], sem
                )
            for r in range(LN):
                pltpu.make_async_copy(
                    vis_v.at[pl.ds(r * S, S)], vis_hbm.at[base + r], sem
                ).wait()

    return chase


def _make_gather(R, E, NROW):
    W = NC * NS
    PERW = R // W
    CH = 128
    NCHUNK = PERW // CH
    NBUF = 4
    LAG = 2
    TROW = NROW - 1
    STG = TROW // NS
    mesh = plsc.VectorSubcoreMesh(core_axis_name="c", subcore_axis_name="s")

    @functools.partial(
        pl.kernel,
        mesh=mesh,
        out_type=jax.ShapeDtypeStruct((R, E), jnp.float32),
        scratch_types=[
            pltpu.VMEM((PERW // 128, 128), jnp.int32),
            pltpu.VMEM((NBUF, CH, E), jnp.float32),
            pltpu.VMEM_SHARED((NROW, E), jnp.float32),
            [pltpu.SemaphoreType.DMA] * NBUF,
            [pltpu.SemaphoreType.DMA] * NBUF,
        ],
    )
    def gather(idx_hbm, tab_hbm, out_hbm, idx_v, rows_v, tab_s, gsem, ssem):
        sid = lax.axis_index("s")
        wid = sid * NC + lax.axis_index("c")
        base = wid * PERW

        pltpu.sync_copy(
            tab_hbm.at[pl.ds(sid * STG, STG), :], tab_s.at[pl.ds(sid * STG, STG), :]
        )

        @pl.when(sid == 0)
        def _():
            pltpu.sync_copy(
                tab_hbm.at[pl.ds(0, 1), :], tab_s.at[pl.ds(TROW, 1), :]
            )

        NR128 = PERW // 128
        pltpu.sync_copy(idx_hbm.at[pl.ds(wid * NR128, NR128), :], idx_v)
        plsc.subcore_barrier()

        PR = 128 // CH

        def idxsl(c):
            if PR == 1:
                return idx_v.at[c]
            return idx_v.at[c // PR, pl.ds((c % PR) * CH, CH)]

        def gat(c, b):
            pltpu.async_copy(tab_s.at[idxsl(c)], rows_v.at[b], gsem[b])

        def scat(c, b):
            pltpu.async_copy(
                rows_v.at[b], out_hbm.at[pl.ds(base + c * CH, CH), :], ssem[b]
            )

        for b in range(NBUF):
            gat(b, b)

        def body(i, carry):
            for b in range(NBUF):
                c = i * NBUF + b
                pltpu.make_async_copy(
                    tab_s.at[idxsl(c)], rows_v.at[b], gsem[b]
                ).wait()
                scat(c, b)
                r = c + NBUF - LAG
                bb = (b + NBUF - LAG) % NBUF

                @pl.when(jnp.logical_and(r >= NBUF, r < NCHUNK))
                def _():
                    pltpu.make_async_copy(
                        rows_v.at[bb],
                        out_hbm.at[pl.ds(base + (r - NBUF) * CH, CH), :],
                        ssem[bb],
                    ).wait()
                    gat(r, bb)
            return carry

        lax.fori_loop(0, NCHUNK // NBUF, body, 0)

        for b in range(NBUF):
            c = NCHUNK - NBUF + b
            pltpu.make_async_copy(
                rows_v.at[b % NBUF],
                out_hbm.at[pl.ds(base + c * CH, CH), :],
                ssem[b % NBUF],
            ).wait()

    return gather


def _matmul(xt, Wm, B, S, E):
    NB = 8

    def mm(x_ref, w_ref, o_ref):
        for j in range(NB):
            o_ref[j] = lax.dot_general(
                x_ref[j],
                w_ref[...],
                dimension_numbers=(((0,), (0,)), ((), ())),
                preferred_element_type=jnp.float32,
            )

    return pl.pallas_call(
        mm,
        grid=(B // NB,),
        in_specs=[
            pl.BlockSpec((NB, 2, S), lambda i: (i, 0, 0)),
            pl.BlockSpec((2, E), lambda i: (0, 0)),
        ],
        out_specs=pl.BlockSpec((NB, S, E), lambda i: (i, 0, 0)),
        out_shape=jax.ShapeDtypeStruct((B, S, E), jnp.float32),
    )(xt, Wm)


def kernel(x, solutions, W, pattern):
    solutions = solutions.astype(jnp.int32)
    B, S = solutions.shape
    E = pattern.shape[1]
    R = B * S

    zer = jnp.zeros((B, S), jnp.int32)
    visited_time = _make_chase(B, S)(solutions, zer)

    idx2d = visited_time.reshape(R // 128, 128)
    pos_enc = _make_gather(R, E, S + 1)(idx2d, pattern).reshape(B, S, E)

    x_embedding = _matmul(jnp.swapaxes(x, 1, 2), W, B, S, E)

    return (x_embedding, pos_enc, visited_time)


# speedup vs baseline: 1.0548x; 1.0548x over previous
import functools

import jax
import jax.numpy as jnp
from jax import lax
from jax.experimental import pallas as pl
from jax.experimental.pallas import tpu as pltpu
from jax.experimental.pallas import tpu_sc as plsc

NC = 2
NS = 16
LN = 16


def _make_chase(B, S):
    G = B // LN
    mesh = plsc.VectorSubcoreMesh(core_axis_name="c", subcore_axis_name="s")

    @functools.partial(
        pl.kernel,
        mesh=mesh,
        out_type=jax.ShapeDtypeStruct((B, S), jnp.int32),
        scratch_types=[
            pltpu.VMEM((LN * S,), jnp.int32),
            pltpu.VMEM((LN * S,), jnp.int32),
            pltpu.SemaphoreType.DMA,
        ],
        compiler_params=pltpu.CompilerParams(needs_layout_passes=False),
    )
    def chase(sol_hbm, zer_hbm, vis_hbm, sol_v, vis_v, sem):
        wid = lax.axis_index("s") * NC + lax.axis_index("c")

        @pl.when(wid < G)
        def _():
            base = wid * LN
            for r in range(LN):
                pltpu.async_copy(
                    sol_hbm.at[base + r], sol_v.at[pl.ds(r * S, S)], sem
                )
                pltpu.async_copy(
                    zer_hbm.at[base + r], vis_v.at[pl.ds(r * S, S)], sem
                )
            for r in range(LN):
                pltpu.make_async_copy(
                    sol_hbm.at[base + r], sol_v.at[pl.ds(r * S, S)], sem
                ).wait()
                pltpu.make_async_copy(
                    zer_hbm.at[base + r], vis_v.at[pl.ds(r * S, S)], sem
                ).wait()

            rowbase = lax.iota(jnp.int32, LN) * S
            zeros = jnp.zeros((LN,), jnp.int32)
            ones = jnp.full((LN,), 1, jnp.int32)

            def step(i, carry):
                del i
                pre, stepv = carry
                cur = plsc.load_gather(sol_v, [rowbase + pre])
                plsc.store_scatter(vis_v, [rowbase + cur], stepv)
                return (cur, stepv + ones)

            lax.fori_loop(0, S, step, (zeros, ones), unroll=8)

            for r in range(LN):
                pltpu.async_copy(
                    vis_v.at[pl.ds(r * S, S)], vis_hbm.at[base + r], sem
                )
            for r in range(LN):
                pltpu.make_async_copy(
                    vis_v.at[pl.ds(r * S, S)], vis_hbm.at[base + r], sem
                ).wait()

    return chase


def _make_gather(R, E, NROW):
    W = NC * NS
    PERW = R // W
    CH = 64
    NCHUNK = PERW // CH
    NBUF = 8
    LAG = 2
    TROW = NROW - 1
    STG = TROW // NS
    mesh = plsc.VectorSubcoreMesh(core_axis_name="c", subcore_axis_name="s")

    @functools.partial(
        pl.kernel,
        mesh=mesh,
        out_type=jax.ShapeDtypeStruct((R, E), jnp.float32),
        scratch_types=[
            pltpu.VMEM((PERW // 128, 128), jnp.int32),
            pltpu.VMEM((NBUF, CH, E), jnp.float32),
            pltpu.VMEM_SHARED((NROW, E), jnp.float32),
            [pltpu.SemaphoreType.DMA] * NBUF,
            [pltpu.SemaphoreType.DMA] * NBUF,
        ],
    )
    def gather(idx_hbm, tab_hbm, out_hbm, idx_v, rows_v, tab_s, gsem, ssem):
        sid = lax.axis_index("s")
        wid = sid * NC + lax.axis_index("c")
        base = wid * PERW

        pltpu.sync_copy(
            tab_hbm.at[pl.ds(sid * STG, STG), :], tab_s.at[pl.ds(sid * STG, STG), :]
        )

        @pl.when(sid == 0)
        def _():
            pltpu.sync_copy(
                tab_hbm.at[pl.ds(0, 1), :], tab_s.at[pl.ds(TROW, 1), :]
            )

        NR128 = PERW // 128
        pltpu.sync_copy(idx_hbm.at[pl.ds(wid * NR128, NR128), :], idx_v)
        plsc.subcore_barrier()

        PR = 128 // CH

        def idxsl(c):
            if PR == 1:
                return idx_v.at[c]
            return idx_v.at[c // PR, pl.ds((c % PR) * CH, CH)]

        def gat(c, b):
            pltpu.async_copy(tab_s.at[idxsl(c)], rows_v.at[b], gsem[b])

        def scat(c, b):
            pltpu.async_copy(
                rows_v.at[b], out_hbm.at[pl.ds(base + c * CH, CH), :], ssem[b]
            )

        for b in range(NBUF):
            gat(b, b)

        def body(i, carry):
            for b in range(NBUF):
                c = i * NBUF + b
                pltpu.make_async_copy(
                    tab_s.at[idxsl(c)], rows_v.at[b], gsem[b]
                ).wait()
                scat(c, b)
                r = c + NBUF - LAG
                bb = (b + NBUF - LAG) % NBUF

                @pl.when(jnp.logical_and(r >= NBUF, r < NCHUNK))
                def _():
                    pltpu.make_async_copy(
                        rows_v.at[bb],
                        out_hbm.at[pl.ds(base + (r - NBUF) * CH, CH), :],
                        ssem[bb],
                    ).wait()
                    gat(r, bb)
            return carry

        lax.fori_loop(0, NCHUNK // NBUF, body, 0)

        for b in range(NBUF):
            c = NCHUNK - NBUF + b
            pltpu.make_async_copy(
                rows_v.at[b % NBUF],
                out_hbm.at[pl.ds(base + c * CH, CH), :],
                ssem[b % NBUF],
            ).wait()

    return gather


def _matmul(xt, Wm, B, S, E):
    NB = 8

    def mm(x_ref, w_ref, o_ref):
        for j in range(NB):
            o_ref[j] = lax.dot_general(
                x_ref[j],
                w_ref[...],
                dimension_numbers=(((0,), (0,)), ((), ())),
                preferred_element_type=jnp.float32,
            )

    return pl.pallas_call(
        mm,
        grid=(B // NB,),
        in_specs=[
            pl.BlockSpec((NB, 2, S), lambda i: (i, 0, 0)),
            pl.BlockSpec((2, E), lambda i: (0, 0)),
        ],
        out_specs=pl.BlockSpec((NB, S, E), lambda i: (i, 0, 0)),
        out_shape=jax.ShapeDtypeStruct((B, S, E), jnp.float32),
    )(xt, Wm)


def kernel(x, solutions, W, pattern):
    solutions = solutions.astype(jnp.int32)
    B, S = solutions.shape
    E = pattern.shape[1]
    R = B * S

    zer = jnp.zeros((B, S), jnp.int32)
    visited_time = _make_chase(B, S)(solutions, zer)

    idx2d = visited_time.reshape(R // 128, 128)
    pos_enc = _make_gather(R, E, S + 1)(idx2d, pattern).reshape(B, S, E)

    x_embedding = _matmul(jnp.swapaxes(x, 1, 2), W, B, S, E)

    return (x_embedding, pos_enc, visited_time)


# speedup vs baseline: 1.0899x; 1.0332x over previous
import functools

import jax
import jax.numpy as jnp
from jax import lax
from jax.experimental import pallas as pl
from jax.experimental.pallas import tpu as pltpu
from jax.experimental.pallas import tpu_sc as plsc

NC = 2
NS = 16
LN = 16


def _make_chase(B, S):
    G = B // LN
    mesh = plsc.VectorSubcoreMesh(core_axis_name="c", subcore_axis_name="s")

    @functools.partial(
        pl.kernel,
        mesh=mesh,
        out_type=jax.ShapeDtypeStruct((B, S), jnp.int32),
        scratch_types=[
            pltpu.VMEM((LN * S,), jnp.int32),
            pltpu.VMEM((LN * S,), jnp.int32),
            pltpu.SemaphoreType.DMA,
        ],
        compiler_params=pltpu.CompilerParams(needs_layout_passes=False),
    )
    def chase(sol_hbm, zer_hbm, vis_hbm, sol_v, vis_v, sem):
        wid = lax.axis_index("s") * NC + lax.axis_index("c")

        @pl.when(wid < G)
        def _():
            base = wid * LN
            for r in range(LN):
                pltpu.async_copy(
                    sol_hbm.at[base + r], sol_v.at[pl.ds(r * S, S)], sem
                )
                pltpu.async_copy(
                    zer_hbm.at[base + r], vis_v.at[pl.ds(r * S, S)], sem
                )
            for r in range(LN):
                pltpu.make_async_copy(
                    sol_hbm.at[base + r], sol_v.at[pl.ds(r * S, S)], sem
                ).wait()
                pltpu.make_async_copy(
                    zer_hbm.at[base + r], vis_v.at[pl.ds(r * S, S)], sem
                ).wait()

            rowbase = lax.iota(jnp.int32, LN) * S
            ones = jnp.full((LN,), 1, jnp.int32)

            def step(i, carry):
                del i
                addr, stepv = carry
                nxt = plsc.load_gather(sol_v, [addr])
                plsc.store_scatter(vis_v, [nxt], stepv)
                return (nxt, stepv + ones)

            lax.fori_loop(0, S, step, (rowbase, ones), unroll=8)

            for r in range(LN):
                pltpu.async_copy(
                    vis_v.at[pl.ds(r * S, S)], vis_hbm.at[base + r], sem
                )
            for r in range(LN):
                pltpu.make_async_copy(
                    vis_v.at[pl.ds(r * S, S)], vis_hbm.at[base + r], sem
                ).wait()

    return chase


def _make_gather(R, E, NROW):
    W = NC * NS
    PERW = R // W
    CH = 64
    NCHUNK = PERW // CH
    NBUF = 8
    LAG = 2
    TROW = NROW - 1
    STG = TROW // NS
    mesh = plsc.VectorSubcoreMesh(core_axis_name="c", subcore_axis_name="s")

    @functools.partial(
        pl.kernel,
        mesh=mesh,
        out_type=jax.ShapeDtypeStruct((R, E), jnp.float32),
        scratch_types=[
            pltpu.VMEM((PERW // 128, 128), jnp.int32),
            pltpu.VMEM((NBUF, CH, E), jnp.float32),
            pltpu.VMEM_SHARED((NROW, E), jnp.float32),
            [pltpu.SemaphoreType.DMA] * NBUF,
            [pltpu.SemaphoreType.DMA] * NBUF,
        ],
    )
    def gather(idx_hbm, tab_hbm, out_hbm, idx_v, rows_v, tab_s, gsem, ssem):
        sid = lax.axis_index("s")
        wid = sid * NC + lax.axis_index("c")
        base = wid * PERW

        pltpu.sync_copy(
            tab_hbm.at[pl.ds(sid * STG, STG), :], tab_s.at[pl.ds(sid * STG, STG), :]
        )

        @pl.when(sid == 0)
        def _():
            pltpu.sync_copy(
                tab_hbm.at[pl.ds(0, 1), :], tab_s.at[pl.ds(TROW, 1), :]
            )

        NR128 = PERW // 128
        pltpu.sync_copy(idx_hbm.at[pl.ds(wid * NR128, NR128), :], idx_v)
        plsc.subcore_barrier()

        PR = 128 // CH

        def idxsl(c):
            if PR == 1:
                return idx_v.at[c]
            return idx_v.at[c // PR, pl.ds((c % PR) * CH, CH)]

        def gat(c, b):
            pltpu.async_copy(tab_s.at[idxsl(c)], rows_v.at[b], gsem[b])

        def scat(c, b):
            pltpu.async_copy(
                rows_v.at[b], out_hbm.at[pl.ds(base + c * CH, CH), :], ssem[b]
            )

        for b in range(NBUF):
            gat(b, b)

        def body(i, carry):
            for b in range(NBUF):
                c = i * NBUF + b
                pltpu.make_async_copy(
                    tab_s.at[idxsl(c)], rows_v.at[b], gsem[b]
                ).wait()
                scat(c, b)
                r = c + NBUF - LAG
                bb = (b + NBUF - LAG) % NBUF

                @pl.when(jnp.logical_and(r >= NBUF, r < NCHUNK))
                def _():
                    pltpu.make_async_copy(
                        rows_v.at[bb],
                        out_hbm.at[pl.ds(base + (r - NBUF) * CH, CH), :],
                        ssem[bb],
                    ).wait()
                    gat(r, bb)
            return carry

        lax.fori_loop(0, NCHUNK // NBUF, body, 0)

        for b in range(NBUF):
            c = NCHUNK - NBUF + b
            pltpu.make_async_copy(
                rows_v.at[b % NBUF],
                out_hbm.at[pl.ds(base + c * CH, CH), :],
                ssem[b % NBUF],
            ).wait()

    return gather


def _matmul(xt, Wm, B, S, E):
    NB = 8

    def mm(x_ref, w_ref, o_ref):
        for j in range(NB):
            o_ref[j] = lax.dot_general(
                x_ref[j],
                w_ref[...],
                dimension_numbers=(((0,), (0,)), ((), ())),
                preferred_element_type=jnp.float32,
            )

    return pl.pallas_call(
        mm,
        grid=(B // NB,),
        in_specs=[
            pl.BlockSpec((NB, 2, S), lambda i: (i, 0, 0)),
            pl.BlockSpec((2, E), lambda i: (0, 0)),
        ],
        out_specs=pl.BlockSpec((NB, S, E), lambda i: (i, 0, 0)),
        out_shape=jax.ShapeDtypeStruct((B, S, E), jnp.float32),
    )(xt, Wm)


def kernel(x, solutions, W, pattern):
    solutions = solutions.astype(jnp.int32)
    B, S = solutions.shape
    E = pattern.shape[1]
    R = B * S

    zer = jnp.zeros((B, S), jnp.int32)
    lane_off = (jnp.arange(B, dtype=jnp.int32) % LN)[:, None] * S
    visited_time = _make_chase(B, S)(solutions + lane_off, zer)

    idx2d = visited_time.reshape(R // 128, 128)
    pos_enc = _make_gather(R, E, S + 1)(idx2d, pattern).reshape(B, S, E)

    x_embedding = _matmul(jnp.swapaxes(x, 1, 2), W, B, S, E)

    return (x_embedding, pos_enc, visited_time)
